# baseline (speedup 1.0000x reference)
import functools

import jax
import jax.numpy as jnp
from jax import lax
from jax.experimental import pallas as pl
from jax.experimental.pallas import tpu as pltpu
from jax.experimental.pallas import tpu_sc as plsc

VOCAB = 1000
N_EMBD = 64
D_PAD = 128
T = 20
B = 1024
NROWS = B * T

_NC = 2
_NS = 16
_NW = _NC * _NS
_RPW = NROWS // _NW


@functools.partial(
    pl.kernel,
    mesh=plsc.VectorSubcoreMesh(core_axis_name="c", subcore_axis_name="s"),
    out_type=jax.ShapeDtypeStruct((NROWS, N_EMBD), jnp.float32),
    scratch_types=[
        pltpu.VMEM((_RPW,), jnp.int32),
        pltpu.VMEM((_RPW, N_EMBD), jnp.float32),
        pltpu.SemaphoreType.DMA,
    ],
    compiler_params=pltpu.CompilerParams(use_tc_tiling_on_sc=False),
)
def _sc_gather(idx_hbm, table_hbm, out_hbm, idx_v, rows_v, sem):
    wid = lax.axis_index("s") * _NC + lax.axis_index("c")
    base = wid * _RPW
    pltpu.sync_copy(idx_hbm.at[pl.ds(base, _RPW)], idx_v)
    pltpu.async_copy(table_hbm.at[idx_v], rows_v, sem).wait()
    pltpu.sync_copy(rows_v, out_hbm.at[pl.ds(base, _RPW)])


_TBLK = 4


def _tc_body(x_ref, pos_ref, w_ref, b_ref, out_ref):
    i = pl.program_id(0)
    for k in range(_TBLK):
        xp = x_ref[k] + pos_ref[i * _TBLK + k, :]
        acc = lax.dot_general(
            w_ref[...], xp,
            dimension_numbers=(((0,), (1,)), ((), ())),
            preferred_element_type=jnp.float32,
        )
        out_ref[k] = acc + b_ref[...]


def _tc_head(x3, pos_pad, W_pad, b_col):
    return pl.pallas_call(
        _tc_body,
        grid=(T // _TBLK,),
        in_specs=[
            pl.BlockSpec((_TBLK, B, N_EMBD), lambda i: (i, 0, 0)),
            pl.BlockSpec((T, N_EMBD), lambda i: (0, 0)),
            pl.BlockSpec((N_EMBD, VOCAB), lambda i: (0, 0)),
            pl.BlockSpec((VOCAB, 1), lambda i: (0, 0)),
        ],
        out_specs=pl.BlockSpec((_TBLK, VOCAB, B), lambda i: (i, 0, 0)),
        out_shape=jax.ShapeDtypeStruct((T, VOCAB, B), jnp.float32),
    )(x3, pos_pad, W_pad, b_col)


def kernel(indices, tok_table, pos_table, W, b):
    Bv, Tv = indices.shape
    idx_tmaj = indices.T.reshape(-1).astype(jnp.int32)
    x2 = _sc_gather(idx_tmaj, tok_table)
    x3 = x2.reshape(Tv, Bv, N_EMBD)
    W_pad = W
    pos_pad = pos_table[:Tv]
    b_col = b.reshape(VOCAB, 1)
    out3 = _tc_head(x3, pos_pad, W_pad, b_col)
    return jnp.transpose(out3, (2, 0, 1))


# speedup vs baseline: 1.1013x; 1.1013x over previous
import functools

import jax
import jax.numpy as jnp
from jax import lax
from jax.experimental import pallas as pl
from jax.experimental.pallas import tpu as pltpu
from jax.experimental.pallas import tpu_sc as plsc

VOCAB = 1000
N_EMBD = 64
D_PAD = 128
T = 20
B = 1024
NROWS = B * T

_NC = 2
_NS = 16
_NW = _NC * _NS
_RPW = NROWS // _NW


@functools.partial(
    pl.kernel,
    mesh=plsc.VectorSubcoreMesh(core_axis_name="c", subcore_axis_name="s"),
    out_type=jax.ShapeDtypeStruct((NROWS, D_PAD), jnp.float32),
    scratch_types=[
        pltpu.VMEM((_RPW,), jnp.int32),
        pltpu.VMEM((_RPW, D_PAD), jnp.float32),
        pltpu.SemaphoreType.DMA,
    ],
)
def _sc_gather(idx_hbm, table_hbm, out_hbm, idx_v, rows_v, sem):
    wid = lax.axis_index("s") * _NC + lax.axis_index("c")
    base = wid * _RPW
    pltpu.sync_copy(idx_hbm.at[pl.ds(base, _RPW)], idx_v)
    pltpu.async_copy(table_hbm.at[idx_v], rows_v, sem).wait()
    pltpu.sync_copy(rows_v, out_hbm.at[pl.ds(base, _RPW)])


_TBLK = 4


def _tc_body(x_ref, pos_ref, w_ref, b_ref, out_ref):
    i = pl.program_id(0)
    for k in range(_TBLK):
        xp = x_ref[k] + pos_ref[i * _TBLK + k, :]
        acc = lax.dot_general(
            w_ref[...], xp,
            dimension_numbers=(((0,), (1,)), ((), ())),
            preferred_element_type=jnp.float32,
        )
        out_ref[k] = acc + b_ref[...]


def _tc_head(x3, pos_pad, W_pad, b_col):
    return pl.pallas_call(
        _tc_body,
        grid=(T // _TBLK,),
        in_specs=[
            pl.BlockSpec((_TBLK, B, D_PAD), lambda i: (i, 0, 0)),
            pl.BlockSpec((T, D_PAD), lambda i: (0, 0)),
            pl.BlockSpec((D_PAD, VOCAB), lambda i: (0, 0)),
            pl.BlockSpec((VOCAB, 1), lambda i: (0, 0)),
        ],
        out_specs=pl.BlockSpec((_TBLK, VOCAB, B), lambda i: (i, 0, 0)),
        out_shape=jax.ShapeDtypeStruct((T, VOCAB, B), jnp.float32),
    )(x3, pos_pad, W_pad, b_col)


def kernel(indices, tok_table, pos_table, W, b):
    Bv, Tv = indices.shape
    idx_tmaj = indices.T.reshape(-1).astype(jnp.int32)
    tok_pad = jnp.pad(tok_table, ((0, 0), (0, D_PAD - N_EMBD)))
    W_pad = jnp.pad(W, ((0, D_PAD - N_EMBD), (0, 0)))
    pos_pad = jnp.pad(pos_table[:Tv], ((0, 0), (0, D_PAD - N_EMBD)))
    x2 = _sc_gather(idx_tmaj, tok_pad)
    x3 = x2.reshape(Tv, Bv, D_PAD)
    b_col = b.reshape(VOCAB, 1)
    out3 = _tc_head(x3, pos_pad, W_pad, b_col)
    return jnp.transpose(out3, (2, 0, 1))
